# baseline (speedup 1.0000x reference)
import jax
import jax.numpy as jnp
from jax import lax
from jax.experimental import pallas as pl
from jax.experimental.pallas import tpu as pltpu
from jax.experimental.pallas import tpu_sc as plsc

N = 10000
E = 320000
T = 3200000
D = 128
B = 9

LANES = 16
BP = 16
TK = 1024
JB = TK // 128
SEG_CHUNK = 2000
N_CHUNKS = E // SEG_CHUNK
CHUNKS_PER_W = N_CHUNKS // 32
VB = E // 128


def _atom_mlp(nf_ref, wa_ref, ba_ref, out_ref):
    out_ref[...] = jax.nn.sigmoid(nf_ref[...] @ wa_ref[...] + ba_ref[...])


def _run_atom_mlp(node_feat, Wa_pad, ba_pad):
    BLK = 2000
    return pl.pallas_call(
        _atom_mlp,
        grid=(N // BLK,),
        in_specs=[
            pl.BlockSpec((BLK, D), lambda i: (i, 0)),
            pl.BlockSpec((D, BP), lambda i: (0, 0)),
            pl.BlockSpec((1, BP), lambda i: (0, 0)),
        ],
        out_specs=pl.BlockSpec((BLK, BP), lambda i: (i, 0)),
        out_shape=jax.ShapeDtypeStruct((N, BP), jnp.float32),
    )(node_feat, Wa_pad, ba_pad)


def _build_v(apad_hbm, gd_hbm, tc_hbm, v_hbm, gdb, tcb, rows, outb, sem):
    wid = lax.axis_index("s") * 2 + lax.axis_index("c")
    nb_w = (VB - wid + 31) // 32
    iota = lax.iota(jnp.int32, LANES)

    def batch_body(k, carry):
        base = (k * 32 + wid) * 128
        pltpu.sync_copy(gd_hbm.at[pl.ds(base, 128)], gdb)
        pltpu.sync_copy(tc_hbm.at[pl.ds(base, 128)], tcb)
        pltpu.async_copy(apad_hbm.at[gdb], rows, sem).wait()

        def grp_body(g, carry2):
            r = g * LANES + iota
            tcv = plsc.load_gather(tcb, [r])
            for col in range(B):
                cv = jnp.full((LANES,), col, jnp.int32)
                vv = plsc.load_gather(rows, [r, cv])
                plsc.store_scatter(outb, [r, cv], vv * tcv)
            return carry2

        lax.fori_loop(0, 128 // LANES, grp_body, 0)
        pltpu.sync_copy(outb, v_hbm.at[pl.ds(base, 128)])
        return carry

    lax.fori_loop(0, nb_w, batch_body, 0)


def _run_build_v(A_pad, graph_dst, three_cutoff):
    mesh = plsc.VectorSubcoreMesh(core_axis_name="c", subcore_axis_name="s")
    f = pl.kernel(
        _build_v,
        out_type=jax.ShapeDtypeStruct((E, BP), jnp.float32),
        mesh=mesh,
        compiler_params=pltpu.CompilerParams(needs_layout_passes=False, use_tc_tiling_on_sc=False),
        scratch_types=[
            pltpu.VMEM((128,), jnp.int32),
            pltpu.VMEM((128,), jnp.float32),
            pltpu.VMEM((128, BP), jnp.float32),
            pltpu.VMEM((128, BP), jnp.float32),
            pltpu.SemaphoreType.DMA,
        ],
    )
    return f(A_pad, graph_dst, three_cutoff)


def _scalar_at(vref, i):
    grp = i // LANES
    off = i % LANES
    v = vref[pl.ds(grp * LANES, LANES)]
    lane = lax.iota(jnp.int32, LANES)
    sel = jnp.where(lane == off, v, 0)
    return lax.reduce_sum_p.bind(sel, axes=(0,))


def _segsum(tb_hbm, ls_hbm, ld_hbm, seg_hbm, tc_hbm, v_hbm, offs_hbm,
            nb_hbm, ls2d, ld2d, segb, basisb, vrows, w1b, accum, offs_vmem,
            sem_i, sem_g):
    wid = lax.axis_index("s") * 2 + lax.axis_index("c")
    iota = lax.iota(jnp.int32, LANES)
    zeros = jnp.zeros((LANES,), jnp.float32)

    pltpu.sync_copy(offs_hbm, offs_vmem)

    for k in range(CHUNKS_PER_W):
        c = wid * CHUNKS_PER_W + k
        segbase = c * SEG_CHUNK
        lo = _scalar_at(offs_vmem, c)
        hi = _scalar_at(offs_vmem, c + 1)

        def zero_body(i, carry):
            for col in range(B):
                accum[col, pl.ds(i * LANES, LANES)] = zeros
            return carry

        lax.fori_loop(0, SEG_CHUNK // LANES, zero_body, 0)

        def block_body(tb, carry):
            base = tb * TK
            cp = []
            for j in range(JB):
                cp.append(pltpu.async_copy(
                    ls_hbm.at[pl.ds(base + j * 128, 128)], ls2d.at[j], sem_i))
                cp.append(pltpu.async_copy(
                    ld_hbm.at[pl.ds(base + j * 128, 128)], ld2d.at[j], sem_i))
            cp.append(pltpu.async_copy(
                seg_hbm.at[pl.ds(base, TK)], segb, sem_i))
            cp.append(pltpu.async_copy(
                tb_hbm.at[pl.ds(base * BP, TK * BP)], basisb, sem_i))
            for h in cp:
                h.wait()

            gp = []
            for j in range(JB):
                gp.append(pltpu.async_copy(
                    v_hbm.at[ld2d.at[j]], vrows.at[j], sem_g))
                gp.append(pltpu.async_copy(
                    tc_hbm.at[ls2d.at[j]], w1b.at[j], sem_g))
            for h in gp:
                h.wait()

            for j in range(JB):
                jv = jnp.full((LANES,), j, jnp.int32)

                def grp(g, carry2, j=j, jv=jv):
                    off = g * LANES
                    row = off + iota
                    seg_v = segb[pl.ds(j * 128 + off, LANES)]
                    w1_v = w1b[j, pl.ds(off, LANES)]
                    segl = seg_v - segbase
                    mask = (segl >= 0) & (segl < SEG_CHUNK)
                    idxc = jnp.clip(segl, 0, SEG_CHUNK - 1)
                    trow = j * 128 + row
                    trow9 = trow * BP
                    for col in range(B):
                        cv = jnp.full((LANES,), col, jnp.int32)
                        bas = plsc.load_gather(basisb, [trow9 + col])
                        vv = plsc.load_gather(vrows, [jv, row, cv])
                        val = bas * vv * w1_v
                        plsc.addupdate_scatter(accum.at[col], [idxc], val,
                                               mask=mask)
                    return carry2

                lax.fori_loop(0, 128 // LANES, grp, 0)
            return carry

        lax.fori_loop(lo // TK, (hi + TK - 1) // TK, block_body, 0)

        for col in range(B):
            pltpu.sync_copy(accum.at[col],
                            nb_hbm.at[col, pl.ds(segbase, SEG_CHUNK)])


def _run_segsum(three_basis, line_src, line_dst, segment_ids, three_cutoff,
                V, offs):
    mesh = plsc.VectorSubcoreMesh(core_axis_name="c", subcore_axis_name="s")
    f = pl.kernel(
        _segsum,
        out_type=jax.ShapeDtypeStruct((B, E), jnp.float32),
        mesh=mesh,
        compiler_params=pltpu.CompilerParams(needs_layout_passes=False, use_tc_tiling_on_sc=False),
        scratch_types=[
            pltpu.VMEM((JB, 128), jnp.int32),
            pltpu.VMEM((JB, 128), jnp.int32),
            pltpu.VMEM((TK,), jnp.int32),
            pltpu.VMEM((TK * BP,), jnp.float32),
            pltpu.VMEM((JB, 128, BP), jnp.float32),
            pltpu.VMEM((JB, 128), jnp.float32),
            pltpu.VMEM((B, SEG_CHUNK), jnp.float32),
            pltpu.VMEM((N_CHUNKS + 8,), jnp.int32),
            pltpu.SemaphoreType.DMA,
            pltpu.SemaphoreType.DMA,
        ],
    )
    return f(three_basis, line_src, line_dst, segment_ids, three_cutoff,
             V, offs)


def _final_stage(nb_ref, ef_ref, wb_ref, bb_ref, wg_ref, bg_ref, out_ref):
    nb = nb_ref[...]
    h = lax.dot_general(nb, wb_ref[...], (((0,), (0,)), ((), ())),
                        preferred_element_type=jnp.float32) + bb_ref[...]
    g = lax.dot_general(nb, wg_ref[...], (((0,), (0,)), ((), ())),
                        preferred_element_type=jnp.float32) + bg_ref[...]
    h = h * jax.nn.sigmoid(h)
    g = jax.nn.sigmoid(g)
    out_ref[...] = ef_ref[...] + h * g


def _run_final(nb_T, edge_feat, Wb, bb, Wg, bg):
    BLK = 2560
    return pl.pallas_call(
        _final_stage,
        grid=(E // BLK,),
        in_specs=[
            pl.BlockSpec((B, BLK), lambda i: (0, i)),
            pl.BlockSpec((BLK, D), lambda i: (i, 0)),
            pl.BlockSpec((B, D), lambda i: (0, 0)),
            pl.BlockSpec((1, D), lambda i: (0, 0)),
            pl.BlockSpec((B, D), lambda i: (0, 0)),
            pl.BlockSpec((1, D), lambda i: (0, 0)),
        ],
        out_specs=pl.BlockSpec((BLK, D), lambda i: (i, 0)),
        out_shape=jax.ShapeDtypeStruct((E, D), jnp.float32),
    )(nb_T, edge_feat, Wb, bb.reshape(1, D), Wg, bg.reshape(1, D))


def kernel(node_feat, edge_feat, three_basis, three_cutoff, graph_dst,
           line_src, line_dst, segment_ids, Wa, ba, Wb, bb, Wg, bg):
    Wa_pad = jnp.pad(Wa, ((0, 0), (0, BP - B)))
    ba_pad = jnp.pad(ba, (0, BP - B)).reshape(1, BP)

    bases = jnp.arange(N_CHUNKS + 1, dtype=jnp.int32) * SEG_CHUNK
    offs = jnp.searchsorted(segment_ids, bases, side="left").astype(jnp.int32)
    offs = jnp.pad(offs, (0, 7), constant_values=T)

    A_pad = _run_atom_mlp(node_feat, Wa_pad, ba_pad)
    V = _run_build_v(A_pad, graph_dst, three_cutoff)
    tb_flat = jnp.pad(three_basis, ((0, 0), (0, BP - B))).reshape(T * BP)
    nb_T = _run_segsum(tb_flat, line_src, line_dst,
                       segment_ids, three_cutoff, V, offs)
    return _run_final(nb_T, edge_feat, Wb, bb, Wg, bg)


# speedup vs baseline: 2.3188x; 2.3188x over previous
import jax
import jax.numpy as jnp
from jax import lax
from jax.experimental import pallas as pl
from jax.experimental.pallas import tpu as pltpu
from jax.experimental.pallas import tpu_sc as plsc

N = 10000
E = 320000
T = 3200000
D = 128
B = 9

LANES = 16
BP = 16
TK = 1024
JB = TK // 128
SEG_CHUNK = 2000
N_CHUNKS = E // SEG_CHUNK
CHUNKS_PER_W = N_CHUNKS // 32
VB = E // 128


def _atom_mlp(nf_ref, wa_ref, ba_ref, out_ref):
    out_ref[...] = jax.nn.sigmoid(nf_ref[...] @ wa_ref[...] + ba_ref[...])


def _run_atom_mlp(node_feat, Wa_pad, ba_pad):
    BLK = 2000
    return pl.pallas_call(
        _atom_mlp,
        grid=(N // BLK,),
        in_specs=[
            pl.BlockSpec((BLK, D), lambda i: (i, 0)),
            pl.BlockSpec((D, BP), lambda i: (0, 0)),
            pl.BlockSpec((1, BP), lambda i: (0, 0)),
        ],
        out_specs=pl.BlockSpec((BLK, BP), lambda i: (i, 0)),
        out_shape=jax.ShapeDtypeStruct((N, BP), jnp.float32),
    )(node_feat, Wa_pad, ba_pad)


def _build_v(apad_hbm, gd_hbm, tc_hbm, v_hbm, gdb, tcb, rows, outb, sem):
    wid = lax.axis_index("s") * 2 + lax.axis_index("c")
    nb_w = (VB - wid + 31) // 32
    iota = lax.iota(jnp.int32, LANES)

    def batch_body(k, carry):
        base = (k * 32 + wid) * 128
        pltpu.sync_copy(gd_hbm.at[pl.ds(base, 128)], gdb)
        pltpu.sync_copy(tc_hbm.at[pl.ds(base, 128)], tcb)
        pltpu.async_copy(apad_hbm.at[gdb], rows, sem).wait()

        def grp_body(g, carry2):
            r = g * LANES + iota
            tcv = plsc.load_gather(tcb, [r])
            for col in range(B):
                cv = jnp.full((LANES,), col, jnp.int32)
                vv = plsc.load_gather(rows, [r, cv])
                plsc.store_scatter(outb, [r, cv], vv * tcv)
            return carry2

        lax.fori_loop(0, 128 // LANES, grp_body, 0)
        pltpu.sync_copy(outb, v_hbm.at[pl.ds(base, 128)])
        return carry

    lax.fori_loop(0, nb_w, batch_body, 0)


def _run_build_v(A_pad, graph_dst, three_cutoff):
    mesh = plsc.VectorSubcoreMesh(core_axis_name="c", subcore_axis_name="s")
    f = pl.kernel(
        _build_v,
        out_type=jax.ShapeDtypeStruct((E, BP), jnp.float32),
        mesh=mesh,
        compiler_params=pltpu.CompilerParams(needs_layout_passes=False, use_tc_tiling_on_sc=False),
        scratch_types=[
            pltpu.VMEM((128,), jnp.int32),
            pltpu.VMEM((128,), jnp.float32),
            pltpu.VMEM((128, BP), jnp.float32),
            pltpu.VMEM((128, BP), jnp.float32),
            pltpu.SemaphoreType.DMA,
        ],
    )
    return f(A_pad, graph_dst, three_cutoff)


TKR = 12800
NJOBS = (T // TKR) * B


def _repack(tbT_hbm, flat_hbm, stage, sem):
    wid = lax.axis_index("s") * 2 + lax.axis_index("c")
    njobs_w = (NJOBS - wid + 31) // 32

    def body(k, carry):
        jj = k * 32 + wid
        b = jj // (T // TKR)
        kk = jj % (T // TKR)
        base = kk * TKR
        pltpu.async_copy(tbT_hbm.at[b, pl.ds(base, TKR)], stage, sem).wait()
        pltpu.async_copy(stage, flat_hbm.at[pl.ds(b * T + base, TKR)],
                         sem).wait()
        return carry

    lax.fori_loop(0, njobs_w, body, 0)


def _run_repack(tbT):
    mesh = plsc.VectorSubcoreMesh(core_axis_name="c", subcore_axis_name="s")
    f = pl.kernel(
        _repack,
        out_type=jax.ShapeDtypeStruct((B * T,), jnp.float32),
        mesh=mesh,
        compiler_params=pltpu.CompilerParams(needs_layout_passes=False),
        scratch_types=[
            pltpu.VMEM((TKR,), jnp.float32),
            pltpu.SemaphoreType.DMA,
        ],
    )
    return f(tbT)


def _scalar_at(vref, i):
    grp = i // LANES
    off = i % LANES
    v = vref[pl.ds(grp * LANES, LANES)]
    lane = lax.iota(jnp.int32, LANES)
    sel = jnp.where(lane == off, v, 0)
    return lax.reduce_sum_p.bind(sel, axes=(0,))


def _segsum(tb_hbm, ls_hbm, ld_hbm, seg_hbm, tc_hbm, v_hbm, offs_hbm,
            nb_hbm, ls2d, ld2d, segb, basisb, vrows, w1b, accum, offs_vmem,
            sem_i, sem_g, sem_s, sem_g2):
    wid = lax.axis_index("s") * 2 + lax.axis_index("c")
    iota = lax.iota(jnp.int32, LANES)
    zeros = jnp.zeros((LANES,), jnp.float32)

    pltpu.sync_copy(offs_hbm, offs_vmem)

    for k in range(CHUNKS_PER_W):
        c = wid * CHUNKS_PER_W + k
        segbase = c * SEG_CHUNK
        lo = _scalar_at(offs_vmem, c)
        hi = _scalar_at(offs_vmem, c + 1)

        def zero_body(i, carry):
            for col in range(B):
                accum[col, pl.ds(i * LANES, LANES)] = zeros
            return carry

        lax.fori_loop(0, SEG_CHUNK // LANES, zero_body, 0)

        def block_body(tb, carry):
            base = tb * TK
            cp = []
            for j in range(JB):
                cp.append(pltpu.async_copy(
                    ls_hbm.at[pl.ds(base + j * 128, 128)], ls2d.at[j], sem_i))
                cp.append(pltpu.async_copy(
                    ld_hbm.at[pl.ds(base + j * 128, 128)], ld2d.at[j], sem_i))
            sp = [pltpu.async_copy(seg_hbm.at[pl.ds(base, TK)], segb, sem_s)]
            for b in range(B):
                sp.append(pltpu.async_copy(
                    tb_hbm.at[pl.ds(b * T + base, TK)], basisb.at[b], sem_s))
            for h in cp:
                h.wait()

            sems = (sem_g, sem_g2)
            gp = [(pltpu.async_copy(v_hbm.at[ld2d.at[0]], vrows.at[0],
                                    sems[0]),
                   pltpu.async_copy(tc_hbm.at[ls2d.at[0]], w1b.at[0],
                                    sems[0]))]
            for h in sp:
                h.wait()

            for j in range(JB):
                if j + 1 < JB:
                    sm = sems[(j + 1) % 2]
                    gp.append((pltpu.async_copy(
                        v_hbm.at[ld2d.at[j + 1]], vrows.at[j + 1], sm),
                        pltpu.async_copy(
                        tc_hbm.at[ls2d.at[j + 1]], w1b.at[j + 1], sm)))
                for h in gp[j]:
                    h.wait()
                jv = jnp.full((LANES,), j, jnp.int32)

                def grp(g, carry2, j=j, jv=jv):
                    off = g * LANES
                    row = off + iota
                    seg_v = segb[pl.ds(j * 128 + off, LANES)]
                    w1_v = w1b[j, pl.ds(off, LANES)]
                    segl = seg_v - segbase
                    mask = (segl >= 0) & (segl < SEG_CHUNK)
                    idxc = jnp.clip(segl, 0, SEG_CHUNK - 1)
                    trow = j * 128 + row
                    for col in range(B):
                        cv = jnp.full((LANES,), col, jnp.int32)
                        bas = basisb[col, pl.ds(j * 128 + off, LANES)]
                        vv = plsc.load_gather(vrows, [jv, row, cv])
                        val = bas * vv * w1_v
                        plsc.addupdate_scatter(accum.at[col], [idxc], val,
                                               mask=mask)
                    return carry2

                lax.fori_loop(0, 128 // LANES, grp, 0)
            return carry

        lax.fori_loop(lo // TK, (hi + TK - 1) // TK, block_body, 0)

        for col in range(B):
            pltpu.sync_copy(accum.at[col],
                            nb_hbm.at[col, pl.ds(segbase, SEG_CHUNK)])


def _run_segsum(three_basis, line_src, line_dst, segment_ids, three_cutoff,
                V, offs):
    mesh = plsc.VectorSubcoreMesh(core_axis_name="c", subcore_axis_name="s")
    f = pl.kernel(
        _segsum,
        out_type=jax.ShapeDtypeStruct((B, E), jnp.float32),
        mesh=mesh,
        compiler_params=pltpu.CompilerParams(needs_layout_passes=False, use_tc_tiling_on_sc=False),
        scratch_types=[
            pltpu.VMEM((JB, 128), jnp.int32),
            pltpu.VMEM((JB, 128), jnp.int32),
            pltpu.VMEM((TK,), jnp.int32),
            pltpu.VMEM((B, TK), jnp.float32),
            pltpu.VMEM((JB, 128, BP), jnp.float32),
            pltpu.VMEM((JB, 128), jnp.float32),
            pltpu.VMEM((B, SEG_CHUNK), jnp.float32),
            pltpu.VMEM((N_CHUNKS + 8,), jnp.int32),
            pltpu.SemaphoreType.DMA,
            pltpu.SemaphoreType.DMA,
            pltpu.SemaphoreType.DMA,
            pltpu.SemaphoreType.DMA,
        ],
    )
    return f(three_basis, line_src, line_dst, segment_ids, three_cutoff,
             V, offs)


def _final_stage(nb_ref, ef_ref, wb_ref, bb_ref, wg_ref, bg_ref, out_ref):
    nb = nb_ref[...]
    h = lax.dot_general(nb, wb_ref[...], (((0,), (0,)), ((), ())),
                        preferred_element_type=jnp.float32) + bb_ref[...]
    g = lax.dot_general(nb, wg_ref[...], (((0,), (0,)), ((), ())),
                        preferred_element_type=jnp.float32) + bg_ref[...]
    h = h * jax.nn.sigmoid(h)
    g = jax.nn.sigmoid(g)
    out_ref[...] = ef_ref[...] + h * g


def _run_final(nb_T, edge_feat, Wb, bb, Wg, bg):
    BLK = 2560
    return pl.pallas_call(
        _final_stage,
        grid=(E // BLK,),
        in_specs=[
            pl.BlockSpec((B, BLK), lambda i: (0, i)),
            pl.BlockSpec((BLK, D), lambda i: (i, 0)),
            pl.BlockSpec((B, D), lambda i: (0, 0)),
            pl.BlockSpec((1, D), lambda i: (0, 0)),
            pl.BlockSpec((B, D), lambda i: (0, 0)),
            pl.BlockSpec((1, D), lambda i: (0, 0)),
        ],
        out_specs=pl.BlockSpec((BLK, D), lambda i: (i, 0)),
        out_shape=jax.ShapeDtypeStruct((E, D), jnp.float32),
    )(nb_T, edge_feat, Wb, bb.reshape(1, D), Wg, bg.reshape(1, D))


def kernel(node_feat, edge_feat, three_basis, three_cutoff, graph_dst,
           line_src, line_dst, segment_ids, Wa, ba, Wb, bb, Wg, bg):
    Wa_pad = jnp.pad(Wa, ((0, 0), (0, BP - B)))
    ba_pad = jnp.pad(ba, (0, BP - B)).reshape(1, BP)

    bases = jnp.arange(N_CHUNKS + 1, dtype=jnp.int32) * SEG_CHUNK
    offs = jnp.searchsorted(segment_ids, bases, side="left").astype(jnp.int32)
    offs = jnp.pad(offs, (0, 7), constant_values=T)

    A_pad = _run_atom_mlp(node_feat, Wa_pad, ba_pad)
    V = _run_build_v(A_pad, graph_dst, three_cutoff)
    tb_flat = _run_repack(three_basis.T)
    nb_T = _run_segsum(tb_flat, line_src, line_dst,
                       segment_ids, three_cutoff, V, offs)
    return _run_final(nb_T, edge_feat, Wb, bb, Wg, bg)
